# baseline (speedup 1.0000x reference)
import dataclasses

import jax
import jax.numpy as jnp
from jax import lax
from jax.experimental import pallas as pl
from jax.experimental.pallas import tpu as pltpu
from jax.experimental.pallas import tpu_sc as plsc

_N = 1000
_NP = 1024
_E = 1200
_DIN = 128
_H = 256
_L = 128

_NC = 2
_NS = 16
_EC = 48
_NT = _E // _EC

_f32 = jnp.float32
_bf16 = jnp.bfloat16


def _sc_mesh():
    return plsc.VectorSubcoreMesh(core_axis_name="c", subcore_axis_name="s")


def _sc_cp():
    cp = pltpu.CompilerParams()
    if "needs_layout_passes" in pltpu.CompilerParams.__dataclass_fields__:
        cp = dataclasses.replace(cp, needs_layout_passes=False)
    return cp


def _tc_a_body(x_ref, dst_row_ref, W_enc_ref, b_enc_ref, W_g1_ref,
               g1_ref, dinv_ref):
    dst_row = dst_row_ref[...]
    ioNE = lax.broadcasted_iota(jnp.int32, (_N, _E), 0)
    ohT_dst = (ioNE == dst_row).astype(_f32)
    deg = jnp.sum(ohT_dst, axis=1, keepdims=True) + 1.0
    dinv = lax.rsqrt(jnp.maximum(deg, 1.0))
    dinv_ref[...] = dinv

    h0 = jax.nn.relu(jnp.dot(x_ref[...], W_enc_ref[...],
                             preferred_element_type=_f32) + b_enc_ref[...])
    t1 = jnp.dot(h0, W_g1_ref[...], preferred_element_type=_f32)
    g1 = (dinv * t1).astype(_bf16)
    g1_ref[...] = jnp.concatenate(
        [g1, jnp.zeros((_NP - _N, _H), _bf16)], axis=0)


def _sc_gather_body(g_hbm, src_hbm, rows_hbm, sidx_v, rows_v, sem):
    c = lax.axis_index("c")
    s = lax.axis_index("s")
    wid = c * _NS + s

    @pl.when(wid < _NT)
    def _():
        base = wid * _EC
        pltpu.sync_copy(src_hbm.at[pl.ds(base, _EC)], sidx_v)
        pltpu.async_copy(g_hbm.at[sidx_v], rows_v, sem).wait()
        pltpu.sync_copy(rows_v, rows_hbm.at[pl.ds(base, _EC)])


def _sc_gather(wordw, g_pad_f32view, src):
    kfn = pl.kernel(
        _sc_gather_body,
        out_type=jax.ShapeDtypeStruct((_E, wordw), _f32),
        mesh=_sc_mesh(),
        scratch_types=[
            pltpu.VMEM((_EC,), jnp.int32),
            pltpu.VMEM((_EC, wordw), _f32),
            pltpu.SemaphoreType.DMA,
        ],
    )
    return kfn(g_pad_f32view, src)


def _tc_c_body(rows_ref, dst_row_ref, g1_ref, dinv_ref, b_g1_ref, W_g2_ref,
               g2_ref):
    dst_row = dst_row_ref[...]
    ioNE = lax.broadcasted_iota(jnp.int32, (_N, _E), 0)
    ohT = (ioNE == dst_row).astype(_f32).astype(_bf16)
    agg = jnp.dot(ohT, rows_ref[...], preferred_element_type=_f32)
    dinv = dinv_ref[...]
    g1n = g1_ref[pl.ds(0, _N), :].astype(_f32)
    h1 = jax.nn.relu(dinv * (agg + g1n) + b_g1_ref[...])
    t2 = jnp.dot(h1, W_g2_ref[...], preferred_element_type=_f32)
    g2 = dinv * t2
    g2_ref[...] = jnp.concatenate(
        [g2, jnp.zeros((_NP - _N, _L), _f32)], axis=0)


def _tc_e_body(rows_ref, dst_row_ref, g2_ref, dinv_ref, b_g2_ref,
               w_head_ref, b_sw_ref, b_v_ref,
               ssrc_ref, sdst_ref, vw_ref):
    dst_row = dst_row_ref[...]
    ioNE = lax.broadcasted_iota(jnp.int32, (_N, _E), 0)
    ohT = (ioNE == dst_row).astype(_f32).astype(_bf16)
    agg = jnp.dot(ohT, rows_ref[...], preferred_element_type=_f32)
    dinv = dinv_ref[...]
    g2n = g2_ref[pl.ds(0, _N), :]
    h2 = jax.nn.relu(dinv * (agg + g2n) + b_g2_ref[...])
    sv = jnp.dot(h2, w_head_ref[...], preferred_element_type=_f32)
    pad = jnp.zeros((_NP - _N, 1), _f32)
    ssrc_ref[...] = jnp.concatenate([sv[:, 0:1] + b_sw_ref[...], pad], axis=0)
    sdst_ref[...] = jnp.concatenate([sv[:, 1:2], pad], axis=0)
    vr = jax.nn.sigmoid(sv[:, 2:3] + b_v_ref[...])
    vw_ref[...] = (0.9 + 0.2 * vr) ** 2


def _sc_head_body(ssrc_hbm, sdst_hbm, src_hbm, dst_hbm, yw_hbm,
                  ssrc_v, sdst_v, sidx_v, didx_v, out_v, sem):
    c = lax.axis_index("c")
    s = lax.axis_index("s")
    wid = c * _NS + s

    @pl.when(wid < _NT)
    def _():
        base = wid * _EC
        c1 = pltpu.async_copy(ssrc_hbm, ssrc_v, sem)
        c2 = pltpu.async_copy(sdst_hbm, sdst_v, sem)
        c3 = pltpu.async_copy(src_hbm.at[pl.ds(base, _EC)], sidx_v, sem)
        c4 = pltpu.async_copy(dst_hbm.at[pl.ds(base, _EC)], didx_v, sem)
        c1.wait()
        c2.wait()
        c3.wait()
        c4.wait()
        for j in range(_EC // 16):
            si = sidx_v[pl.ds(j * 16, 16)]
            di = didx_v[pl.ds(j * 16, 16)]
            a = plsc.load_gather(ssrc_v, [si])
            d = plsc.load_gather(sdst_v, [di])
            z = a + d
            out_v[pl.ds(j * 16, 16)] = 1.0 / (1.0 + jnp.exp(-z))
        pltpu.sync_copy(out_v, yw_hbm.at[pl.ds(base, _EC)])


def _sc_head(ssrc, sdst, src, dst):
    kfn = pl.kernel(
        _sc_head_body,
        compiler_params=_sc_cp(),
        out_type=jax.ShapeDtypeStruct((_E,), _f32),
        mesh=_sc_mesh(),
        scratch_types=[
            pltpu.VMEM((_NP,), _f32),
            pltpu.VMEM((_NP,), _f32),
            pltpu.VMEM((_EC,), jnp.int32),
            pltpu.VMEM((_EC,), jnp.int32),
            pltpu.VMEM((_EC,), _f32),
            pltpu.SemaphoreType.DMA,
        ],
    )
    return kfn(ssrc, sdst, src, dst)


def kernel(x, edge_index, W_enc, b_enc, W_g1, b_g1, W_g2, b_g2, W_sw, b_sw,
           W_v, b_v, cvx_p_inj, cvx_q_inj, cvx_y0, cvx_r_pu, cvx_x_pu,
           cvx_bigM_flow, cvx_bigM_v, cvx_A_from, cvx_A_to, cvx_sub_mask,
           cvx_non_sub_mask, cvx_bigM_flow_sq, cvx_z_line_sq):
    src = edge_index[0]
    dst = edge_index[1]
    dst_row = dst.reshape(1, _E)

    g1p, dinv = pl.pallas_call(
        _tc_a_body,
        out_shape=[
            jax.ShapeDtypeStruct((_NP, _H), _bf16),
            jax.ShapeDtypeStruct((_N, 1), _f32),
        ],
    )(x, dst_row, W_enc, b_enc.reshape(1, _H), W_g1)

    g1v = jax.lax.bitcast_convert_type(
        g1p.reshape(_NP, _H // 2, 2), _f32)
    rows1 = _sc_gather(_H // 2, g1v, src)
    rows1_bf = jax.lax.bitcast_convert_type(rows1, _bf16).reshape(_E, _H)

    g2p = pl.pallas_call(
        _tc_c_body,
        out_shape=jax.ShapeDtypeStruct((_NP, _L), _f32),
    )(rows1_bf, dst_row, g1p, dinv, b_g1.reshape(1, _H), W_g2)

    rows2_bf = _sc_gather(_L, g2p, src)

    ssrc, sdst, vw = pl.pallas_call(
        _tc_e_body,
        out_shape=[
            jax.ShapeDtypeStruct((_NP, 1), _f32),
            jax.ShapeDtypeStruct((_NP, 1), _f32),
            jax.ShapeDtypeStruct((_N, 1), _f32),
        ],
    )(rows2_bf, dst_row, g2p, dinv, b_g2.reshape(1, _L),
      jnp.concatenate([W_sw[:_L], W_sw[_L:], W_v], axis=1),
      b_sw.reshape(1, 1), b_v.reshape(1, 1))

    yw = _sc_head(ssrc.reshape(_NP), sdst.reshape(_NP), src, dst)
    return yw, vw[:, 0]


# speedup vs baseline: 1.4551x; 1.4551x over previous
import dataclasses

import jax
import jax.numpy as jnp
from jax import lax
from jax.experimental import pallas as pl
from jax.experimental.pallas import tpu as pltpu
from jax.experimental.pallas import tpu_sc as plsc

_N = 1000
_NP = 1024
_E = 1200
_DIN = 128
_H = 256
_L = 128

_NC = 2
_NS = 16
_EC = 48
_NT = _E // _EC

_f32 = jnp.float32
_bf16 = jnp.bfloat16


def _tc_body(x_ref, src_row_ref, dst_row_ref,
             W_enc_ref, b_enc_ref, W_g1_ref, b_g1_ref, W_g2_ref, b_g2_ref,
             w_head_ref, b_sw_ref, b_v_ref,
             ssrc_ref, sdst_ref, vw_ref):
    dst_row = dst_row_ref[...]
    src_row = src_row_ref[...]

    ioNE = lax.broadcasted_iota(jnp.int32, (_N, _E), 0)
    ohT_dst_f = (ioNE == dst_row).astype(_f32)
    ohT_dst = ohT_dst_f.astype(_bf16)
    ohT_src = (ioNE == src_row).astype(_f32).astype(_bf16)

    deg = jnp.sum(ohT_dst_f, axis=1, keepdims=True) + 1.0
    dinv = lax.rsqrt(jnp.maximum(deg, 1.0))

    M = lax.dot_general(ohT_dst, ohT_src, (((1,), (1,)), ((), ())),
                        preferred_element_type=_f32)

    def conv(t):
        g = dinv * t
        agg = jnp.dot(M, g, preferred_element_type=_f32)
        return dinv * (agg + g)

    x = x_ref[...]
    h0 = jax.nn.relu(jnp.dot(x, W_enc_ref[...], preferred_element_type=_f32)
                     + b_enc_ref[...])
    t1 = jnp.dot(h0, W_g1_ref[...], preferred_element_type=_f32)
    h1 = jax.nn.relu(conv(t1) + b_g1_ref[...])
    t2 = jnp.dot(h1, W_g2_ref[...], preferred_element_type=_f32)
    h2 = jax.nn.relu(conv(t2) + b_g2_ref[...])

    sv = jnp.dot(h2, w_head_ref[...], preferred_element_type=_f32)
    pad = jnp.zeros((_NP - _N, 1), _f32)
    ssrc_ref[...] = jnp.concatenate([sv[:, 0:1] + b_sw_ref[...], pad], axis=0)
    sdst_ref[...] = jnp.concatenate([sv[:, 1:2], pad], axis=0)

    vr = jax.nn.sigmoid(sv[:, 2:3] + b_v_ref[...])
    vw_ref[...] = (0.9 + 0.2 * vr) ** 2


def _sc_head_body(ssrc_hbm, sdst_hbm, srcp_hbm, dstp_hbm, yw_hbm,
                  ssrc_v, sdst_v, sidx_v, didx_v, out_v, sem):
    c = lax.axis_index("c")
    s = lax.axis_index("s")
    wid = c * _NS + s

    @pl.when(wid < _NT)
    def _():
        base = wid * _EC
        c1 = pltpu.async_copy(ssrc_hbm, ssrc_v, sem)
        c2 = pltpu.async_copy(sdst_hbm, sdst_v, sem)
        c3 = pltpu.async_copy(srcp_hbm.at[pl.ds(base, _EC)], sidx_v, sem)
        c4 = pltpu.async_copy(dstp_hbm.at[pl.ds(base, _EC)], didx_v, sem)
        c1.wait()
        c2.wait()
        c3.wait()
        c4.wait()
        for j in range(_EC // 16):
            si = sidx_v[pl.ds(j * 16, 16)]
            di = didx_v[pl.ds(j * 16, 16)]
            a = plsc.load_gather(ssrc_v, [si])
            d = plsc.load_gather(sdst_v, [di])
            z = a + d
            out_v[pl.ds(j * 16, 16)] = 1.0 / (1.0 + jnp.exp(-z))
        pltpu.sync_copy(out_v, yw_hbm.at[pl.ds(base, _EC)])


def _sc_head(ssrc, sdst, src_pad, dst_pad):
    mesh = plsc.VectorSubcoreMesh(core_axis_name="c", subcore_axis_name="s")
    cp = pltpu.CompilerParams()
    if "needs_layout_passes" in pltpu.CompilerParams.__dataclass_fields__:
        cp = dataclasses.replace(cp, needs_layout_passes=False)
    kfn = pl.kernel(
        _sc_head_body,
        compiler_params=cp,
        out_type=jax.ShapeDtypeStruct((_E,), _f32),
        mesh=mesh,
        scratch_types=[
            pltpu.VMEM((_NP,), _f32),
            pltpu.VMEM((_NP,), _f32),
            pltpu.VMEM((_EC,), jnp.int32),
            pltpu.VMEM((_EC,), jnp.int32),
            pltpu.VMEM((_EC,), _f32),
            pltpu.SemaphoreType.DMA,
        ],
    )
    return kfn(ssrc, sdst, src_pad, dst_pad)


def kernel(x, edge_index, W_enc, b_enc, W_g1, b_g1, W_g2, b_g2, W_sw, b_sw,
           W_v, b_v, cvx_p_inj, cvx_q_inj, cvx_y0, cvx_r_pu, cvx_x_pu,
           cvx_bigM_flow, cvx_bigM_v, cvx_A_from, cvx_A_to, cvx_sub_mask,
           cvx_non_sub_mask, cvx_bigM_flow_sq, cvx_z_line_sq):
    src = edge_index[0]
    dst = edge_index[1]
    ssrc, sdst, vw = pl.pallas_call(
        _tc_body,
        out_shape=[
            jax.ShapeDtypeStruct((_NP, 1), _f32),
            jax.ShapeDtypeStruct((_NP, 1), _f32),
            jax.ShapeDtypeStruct((_N, 1), _f32),
        ],
    )(x, src.reshape(1, _E), dst.reshape(1, _E),
      W_enc, b_enc.reshape(1, _H),
      W_g1, b_g1.reshape(1, _H),
      W_g2, b_g2.reshape(1, _L),
      jnp.concatenate([W_sw[:_L], W_sw[_L:], W_v], axis=1),
      b_sw.reshape(1, 1), b_v.reshape(1, 1))

    yw = _sc_head(ssrc.reshape(_NP), sdst.reshape(_NP), src, dst)
    return yw, vw[:, 0]
